# baseline (speedup 1.0000x reference)
import numpy as np
import jax
import jax.numpy as jnp
from jax.experimental import pallas as pl

_C, _T, _H, _W = 3, 32, 224, 224
_TS = _T // 4
_IDX = tuple(int(v) for v in np.linspace(0.0, _T - 1, _TS))


def _body(in_ref, fast_ref, slow_ref):
    x = in_ref[...]
    fast_ref[...] = x
    for k, v in enumerate(_IDX):
        slow_ref[:, k] = x[:, v]


_HS = 2
_HB = _H // _HS

_pack = pl.pallas_call(
    _body,
    grid=(_C, _HS),
    out_shape=(
        jax.ShapeDtypeStruct((_C, _T, _H, _W), jnp.float32),
        jax.ShapeDtypeStruct((_C, _TS, _H, _W), jnp.float32),
    ),
    in_specs=[pl.BlockSpec((1, _T, _HB, _W), lambda c, h: (c, 0, h, 0))],
    out_specs=(
        pl.BlockSpec((1, _T, _HB, _W), lambda c, h: (c, 0, h, 0)),
        pl.BlockSpec((1, _TS, _HB, _W), lambda c, h: (c, 0, h, 0)),
    ),
)


def kernel(frames):
    fast, slow = _pack(frames)
    return (slow, fast)


# speedup vs baseline: 1.0071x; 1.0071x over previous
import numpy as np
import jax
import jax.numpy as jnp
from jax.experimental import pallas as pl

_C, _T, _H, _W = 3, 32, 224, 224
_TS = _T // 4
_IDX = tuple(int(v) for v in np.linspace(0.0, _T - 1, _TS))


_WS = 2
_TB = _T // _WS
_SB = _TS // _WS
_LOCAL = tuple(
    tuple(v - k * _TB for v in _IDX if k * _TB <= v < (k + 1) * _TB)
    for k in range(_WS)
)


def _body(in_ref, fast_ref, slow_ref):
    x = in_ref[...]
    fast_ref[...] = x
    k = pl.program_id(0) % _WS
    for j in range(_SB):
        off = _LOCAL[0][j] + k * (_LOCAL[1][j] - _LOCAL[0][j])
        slow_ref[:, j] = in_ref[0, pl.ds(off, 1)]


_pack = pl.pallas_call(
    _body,
    grid=(_C * _WS,),
    out_shape=(
        jax.ShapeDtypeStruct((_C * _WS, _TB, _H, _W), jnp.float32),
        jax.ShapeDtypeStruct((_C * _WS, _SB, _H, _W), jnp.float32),
    ),
    in_specs=[pl.BlockSpec((1, _TB, _H, _W), lambda w: (w, 0, 0, 0))],
    out_specs=(
        pl.BlockSpec((1, _TB, _H, _W), lambda w: (w, 0, 0, 0)),
        pl.BlockSpec((1, _SB, _H, _W), lambda w: (w, 0, 0, 0)),
    ),
)


def kernel(frames):
    fast, slow = _pack(frames.reshape(_C * _WS, _TB, _H, _W))
    return (slow.reshape(_C, _TS, _H, _W), fast.reshape(_C, _T, _H, _W))
